# baseline (speedup 1.0000x reference)
import math

import jax
import jax.numpy as jnp
from jax import lax
from jax.experimental import pallas as pl
from jax.experimental.pallas import tpu as pltpu
from jax.experimental.pallas import tpu_sc as plsc

D = 128
PAD = 5000
SCALE = math.sqrt(float(D))
NC, NS, LANES = 2, 16, 16
NW = NC * NS
C = 128


def _body(x_hbm, m_hbm, i_hbm, pe_hbm, out_hbm, idx_v, msk_v, x_v, rows_v, sem):
    n = x_hbm.shape[0]
    rows_per_w = n // NW
    steps = rows_per_w // C
    wid = lax.axis_index("s") * NC + lax.axis_index("c")
    base0 = wid * rows_per_w

    @pl.loop(0, steps)
    def _step(t):
        base = base0 + t * C
        pltpu.sync_copy(i_hbm.at[pl.ds(base, C)], idx_v)
        pltpu.sync_copy(m_hbm.at[pl.ds(base, C)], msk_v)
        for j in range(C // LANES):
            sl = pl.ds(j * LANES, LANES)
            iv = idx_v[sl]
            mv = msk_v[sl]
            idx_v[sl] = jnp.where(mv != 0, PAD, jnp.minimum(iv, PAD))
        gather = pltpu.async_copy(pe_hbm.at[idx_v], rows_v, sem)
        pltpu.sync_copy(x_hbm.at[pl.ds(base, C), :], x_v)
        gather.wait()

        @pl.loop(0, C)
        def _row(i):
            for j in range(D // LANES):
                sl = pl.ds(j * LANES, LANES)
                rows_v[i, sl] = SCALE * x_v[i, sl] + rows_v[i, sl]

        pltpu.sync_copy(rows_v, out_hbm.at[pl.ds(base, C), :])


def kernel(x, mask, indices, pe):
    b, l, d = x.shape
    n = b * l
    x2 = x.reshape(n, d)
    m2 = mask.reshape(n).astype(jnp.int32)
    i2 = indices.reshape(n).astype(jnp.int32)

    mesh = plsc.VectorSubcoreMesh(core_axis_name="c", subcore_axis_name="s")
    out = pl.kernel(
        _body,
        out_type=jax.ShapeDtypeStruct((n, d), jnp.float32),
        mesh=mesh,
        scratch_types=[
            pltpu.VMEM((C,), jnp.int32),
            pltpu.VMEM((C,), jnp.int32),
            pltpu.VMEM((C, D), jnp.float32),
            pltpu.VMEM((C, D), jnp.float32),
            pltpu.SemaphoreType.DMA,
        ],
    )(x2, m2, i2, pe)
    return out.reshape(b, l, d)


# speedup vs baseline: 18.8631x; 18.8631x over previous
import math

import jax
import jax.numpy as jnp
from jax import lax
from jax.experimental import pallas as pl
from jax.experimental.pallas import tpu as pltpu
from jax.experimental.pallas import tpu_sc as plsc

D = 128
PAD = 5000
SCALE = math.sqrt(float(D))
NC, NS, LANES = 2, 16, 16
NW = NC * NS
C = 128


def _body(x_hbm, m_hbm, i_hbm, pe_hbm, out_hbm,
          idx_v, msk_v, x_v, rows_v, pe_sh, sem):
    n = x_hbm.shape[0]
    rows_per_w = n // NW
    steps = rows_per_w // C
    sid = lax.axis_index("s")
    wid = sid * NC + lax.axis_index("c")
    base0 = wid * rows_per_w

    @pl.when(sid == 0)
    def _stage():
        pltpu.sync_copy(pe_hbm, pe_sh)

    plsc.subcore_barrier()

    @pl.loop(0, steps)
    def _step(t):
        base = base0 + t * C
        pltpu.sync_copy(i_hbm.at[pl.ds(base, C)], idx_v)
        pltpu.sync_copy(m_hbm.at[pl.ds(base, C)], msk_v)
        for j in range(C // LANES):
            sl = pl.ds(j * LANES, LANES)
            iv = idx_v[sl]
            mv = msk_v[sl]
            idx_v[sl] = jnp.where(mv != 0, PAD, jnp.minimum(iv, PAD))
        gather = pltpu.async_copy(pe_sh.at[idx_v], rows_v, sem)
        pltpu.sync_copy(x_hbm.at[pl.ds(base, C), :], x_v)
        gather.wait()

        @pl.loop(0, C)
        def _row(i):
            for j in range(D // LANES):
                sl = pl.ds(j * LANES, LANES)
                rows_v[i, sl] = SCALE * x_v[i, sl] + rows_v[i, sl]

        pltpu.sync_copy(rows_v, out_hbm.at[pl.ds(base, C), :])


def kernel(x, mask, indices, pe):
    b, l, d = x.shape
    n = b * l
    x2 = x.reshape(n, d)
    m2 = mask.reshape(n).astype(jnp.int32)
    i2 = indices.reshape(n).astype(jnp.int32)

    mesh = plsc.VectorSubcoreMesh(core_axis_name="c", subcore_axis_name="s")
    out = pl.kernel(
        _body,
        out_type=jax.ShapeDtypeStruct((n, d), jnp.float32),
        mesh=mesh,
        scratch_types=[
            pltpu.VMEM((C,), jnp.int32),
            pltpu.VMEM((C,), jnp.int32),
            pltpu.VMEM((C, D), jnp.float32),
            pltpu.VMEM((C, D), jnp.float32),
            pltpu.VMEM_SHARED((PAD + 1, D), jnp.float32),
            pltpu.SemaphoreType.DMA,
        ],
    )(x2, m2, i2, pe)
    return out.reshape(b, l, d)


# speedup vs baseline: 36.8363x; 1.9528x over previous
import math

import jax
import jax.numpy as jnp
from jax import lax
from jax.experimental import pallas as pl
from jax.experimental.pallas import tpu as pltpu
from jax.experimental.pallas import tpu_sc as plsc

D = 128
PAD = 5000
SCALE = math.sqrt(float(D))
NC, NS, LANES = 2, 16, 16
NW = NC * NS
C = 128


def _body(x_hbm, m_hbm, i_hbm, pe_hbm, out_hbm,
          idx_v, msk_v, x_v, rows_v, pe_sh,
          sin0, sin1, sg0, sg1, so0, so1):
    sin = (sin0, sin1)
    sg = (sg0, sg1)
    so = (so0, so1)
    n = x_hbm.shape[0]
    rows_per_w = n // NW
    steps = rows_per_w // C
    sid = lax.axis_index("s")
    wid = sid * NC + lax.axis_index("c")
    base0 = wid * rows_per_w

    @pl.when(sid == 0)
    def _stage():
        pltpu.sync_copy(pe_hbm, pe_sh)

    plsc.subcore_barrier()

    def issue_in(p, t):
        base = base0 + t * C
        pltpu.async_copy(i_hbm.at[pl.ds(base, C)], idx_v.at[p], sin[p])
        pltpu.async_copy(m_hbm.at[pl.ds(base, C)], msk_v.at[p], sin[p])
        pltpu.async_copy(x_hbm.at[pl.ds(base, C), :], x_v.at[p], sin[p])

    def drain_in(p):
        pltpu.make_async_copy(i_hbm.at[pl.ds(0, C)], idx_v.at[p], sin[p]).wait()
        pltpu.make_async_copy(m_hbm.at[pl.ds(0, C)], msk_v.at[p], sin[p]).wait()
        pltpu.make_async_copy(x_hbm.at[pl.ds(0, C), :], x_v.at[p], sin[p]).wait()

    def compute_idx(p):
        for j in range(C // LANES):
            sl = pl.ds(j * LANES, LANES)
            iv = idx_v[p, sl]
            mv = msk_v[p, sl]
            idx_v[p, sl] = jnp.where(mv != 0, PAD, jnp.minimum(iv, PAD))

    def issue_gather(p):
        pltpu.async_copy(pe_sh.at[idx_v.at[p]], rows_v.at[p], sg[p])

    def drain_gather(p):
        pltpu.make_async_copy(pe_sh.at[idx_v.at[p]], rows_v.at[p], sg[p]).wait()

    def axpy(p):
        @plsc.parallel_loop(0, C, 1, unroll=2)
        def _row(i):
            for j in range(D // LANES):
                sl = pl.ds(j * LANES, LANES)
                rows_v[p, i, sl] = SCALE * x_v[p, i, sl] + rows_v[p, i, sl]

    def issue_out(p, t):
        base = base0 + t * C
        pltpu.async_copy(rows_v.at[p], out_hbm.at[pl.ds(base, C), :], so[p])

    def drain_out(p):
        pltpu.make_async_copy(rows_v.at[p], out_hbm.at[pl.ds(0, C), :], so[p]).wait()

    issue_in(0, 0)
    drain_in(0)
    compute_idx(0)
    issue_gather(0)
    issue_in(1, 1)

    @pl.loop(0, steps // 2)
    def _pair(u):
        for p in (0, 1):
            t = 2 * u + p
            np_ = 1 - p
            drain_gather(p)
            axpy(p)
            issue_out(p, t)

            @pl.when(t + 1 < steps)
            def _prep_next():
                drain_in(np_)
                compute_idx(np_)

                @pl.when(t >= 1)
                def _free_rows():
                    drain_out(np_)

                issue_gather(np_)

                @pl.when(t + 2 < steps)
                def _prefetch():
                    issue_in(p, t + 2)

    drain_out(0)
    drain_out(1)


def kernel(x, mask, indices, pe):
    b, l, d = x.shape
    n = b * l
    x2 = x.reshape(n, d)
    m2 = mask.reshape(n).astype(jnp.int32)
    i2 = indices.reshape(n).astype(jnp.int32)

    mesh = plsc.VectorSubcoreMesh(core_axis_name="c", subcore_axis_name="s")
    out = pl.kernel(
        _body,
        out_type=jax.ShapeDtypeStruct((n, d), jnp.float32),
        mesh=mesh,
        scratch_types=[
            pltpu.VMEM((2, C), jnp.int32),
            pltpu.VMEM((2, C), jnp.int32),
            pltpu.VMEM((2, C, D), jnp.float32),
            pltpu.VMEM((2, C, D), jnp.float32),
            pltpu.VMEM_SHARED((PAD + 1, D), jnp.float32),
            pltpu.SemaphoreType.DMA,
            pltpu.SemaphoreType.DMA,
            pltpu.SemaphoreType.DMA,
            pltpu.SemaphoreType.DMA,
            pltpu.SemaphoreType.DMA,
            pltpu.SemaphoreType.DMA,
        ],
    )(x2, m2, i2, pe)
    return out.reshape(b, l, d)


# speedup vs baseline: 42.1642x; 1.1446x over previous
import math

import jax
import jax.numpy as jnp
from jax import lax
from jax.experimental import pallas as pl
from jax.experimental.pallas import tpu as pltpu
from jax.experimental.pallas import tpu_sc as plsc

D = 128
PAD = 5000
SCALE = math.sqrt(float(D))
NC, NS, LANES = 2, 16, 16
NW = NC * NS
G = 80
KG = 2
C = G * KG


def _body(x_hbm, m_hbm, i_hbm, pe_hbm, out_hbm,
          idx0, idx1, idx2, msk0, msk1, msk2, xv0, xv1, xv2, pe_sh,
          sin0, sin1, sin2, sg0, sg1, sg2, so0, so1, so2):
    idx_v = (idx0, idx1, idx2)
    msk_v = (msk0, msk1, msk2)
    x_v = (xv0, xv1, xv2)
    sin = (sin0, sin1, sin2)
    sg = (sg0, sg1, sg2)
    so = (so0, so1, so2)
    n = x_hbm.shape[0]
    rows_per_w = n // NW
    steps = rows_per_w // C
    sid = lax.axis_index("s")
    wid = sid * NC + lax.axis_index("c")
    base0 = wid * rows_per_w

    @pl.when(sid == 0)
    def _stage():
        pltpu.sync_copy(pe_hbm, pe_sh)

    plsc.subcore_barrier()

    def issue_in(p, t):
        base = base0 + t * C
        pltpu.async_copy(i_hbm.at[pl.ds(base, C)], idx_v[p], sin[p])
        pltpu.async_copy(m_hbm.at[pl.ds(base, C)], msk_v[p], sin[p])
        pltpu.async_copy(x_hbm.at[pl.ds(base, C), :], x_v[p], sin[p])

    def drain_in(p):
        pltpu.make_async_copy(i_hbm.at[pl.ds(0, C)], idx_v[p], sin[p]).wait()
        pltpu.make_async_copy(m_hbm.at[pl.ds(0, C)], msk_v[p], sin[p]).wait()
        pltpu.make_async_copy(x_hbm.at[pl.ds(0, C), :], x_v[p], sin[p]).wait()

    def prep(p):
        for j in range(C // LANES):
            sl = pl.ds(j * LANES, LANES)
            iv = idx_v[p][sl]
            mv = msk_v[p][sl]
            idx_v[p][sl] = jnp.where(mv != 0, PAD, jnp.minimum(iv, PAD))
        @plsc.parallel_loop(0, C, 1, unroll=2)
        def _row(i):
            for j in range(D // LANES):
                sl = pl.ds(j * LANES, LANES)
                x_v[p][i, sl] = SCALE * x_v[p][i, sl]

    def issue_gather(p):
        for k in range(KG):
            pltpu.async_copy(pe_sh.at[idx_v[p].at[pl.ds(k * G, G)]],
                             x_v[p].at[pl.ds(k * G, G)], sg[p], add=True)

    def drain_gather(p):
        for k in range(KG):
            pltpu.make_async_copy(pe_sh.at[idx_v[p].at[pl.ds(k * G, G)]],
                                  x_v[p].at[pl.ds(k * G, G)], sg[p]).wait()

    def issue_out(p, t):
        base = base0 + t * C
        pltpu.async_copy(x_v[p], out_hbm.at[pl.ds(base, C), :], so[p])

    def drain_out(p):
        pltpu.make_async_copy(x_v[p], out_hbm.at[pl.ds(0, C), :], so[p]).wait()

    issue_in(0, 0)
    drain_in(0)
    prep(0)
    issue_gather(0)
    issue_in(1, 1)

    def iteration(t, p):
        q = (p + 1) % 3
        r = (p + 2) % 3

        @pl.when(t >= 1)
        def _free():
            drain_out(r)

        @pl.when(t + 2 < steps)
        def _prefetch():
            issue_in(r, t + 2)

        drain_in(q)
        prep(q)
        issue_gather(q)
        drain_gather(p)
        issue_out(p, t)

    @pl.loop(0, steps // 3)
    def _triple(u):
        for e in range(3):
            iteration(3 * u + e, e)

    for t in range(3 * (steps // 3), steps):
        p = t % 3
        drain_out((t + 2) % 3)
        drain_gather(p)
        issue_out(p, t)
    drain_out((steps - 1) % 3)


def kernel(x, mask, indices, pe):
    b, l, d = x.shape
    n = b * l
    x2 = x.reshape(n, d)
    m2 = mask.reshape(n).astype(jnp.int32)
    i2 = indices.reshape(n).astype(jnp.int32)

    mesh = plsc.VectorSubcoreMesh(core_axis_name="c", subcore_axis_name="s")
    out = pl.kernel(
        _body,
        out_type=jax.ShapeDtypeStruct((n, d), jnp.float32),
        mesh=mesh,
        scratch_types=[
            pltpu.VMEM((C,), jnp.int32),
            pltpu.VMEM((C,), jnp.int32),
            pltpu.VMEM((C,), jnp.int32),
            pltpu.VMEM((C,), jnp.int32),
            pltpu.VMEM((C,), jnp.int32),
            pltpu.VMEM((C,), jnp.int32),
            pltpu.VMEM((C, D), jnp.float32),
            pltpu.VMEM((C, D), jnp.float32),
            pltpu.VMEM((C, D), jnp.float32),
            pltpu.VMEM_SHARED((PAD + 1, D), jnp.float32),
            pltpu.SemaphoreType.DMA,
            pltpu.SemaphoreType.DMA,
            pltpu.SemaphoreType.DMA,
            pltpu.SemaphoreType.DMA,
            pltpu.SemaphoreType.DMA,
            pltpu.SemaphoreType.DMA,
            pltpu.SemaphoreType.DMA,
            pltpu.SemaphoreType.DMA,
            pltpu.SemaphoreType.DMA,
        ],
    )(x2, m2, i2, pe)
    return out.reshape(b, l, d)
